# baseline (speedup 1.0000x reference)
import functools

import jax
import jax.numpy as jnp
from jax import lax
from jax.experimental import pallas as pl
from jax.experimental.pallas import tpu as pltpu
from jax.experimental.pallas import tpu_sc as plsc

B, S = 4, 2048
N = B * S
H = 768
NUM_EMB = H // 4
UNI_LAYOUT = NUM_EMB // 2
UNI_TREE = (H - NUM_EMB) // 2
EPS = 1e-6

NC, NS, L = 2, 16, 16
NW = NC * NS
PER_W = N // NW
CHUNK = 8
NCHUNK = PER_W // CHUNK
G = H // L

OFF_MAG = 0
OFF_PRE = OFF_MAG + 12 * NUM_EMB
OFF_TOP = OFF_PRE + 12 * NUM_EMB
OFF_LOW = OFF_TOP + 12 * NUM_EMB
OFF_ROW = OFF_LOW + 12 * NUM_EMB
OFF_COL = OFF_ROW + 257 * UNI_LAYOUT
OFF_TREE = OFF_COL + 257 * UNI_LAYOUT
TABS_LEN = OFF_TREE + 2 * UNI_TREE

_DNUMS = lax.GatherDimensionNumbers(
    offset_dims=(), collapsed_slice_dims=(0,), start_index_map=(0,))


def _lane_bcast(vec, idx):
    return lax.gather(vec, idx[:, None], _DNUMS, slice_sizes=(1,),
                      mode=lax.GatherScatterMode.PROMISE_IN_BOUNDS)


def _sc_partial():
    mesh = plsc.VectorSubcoreMesh(core_axis_name="c", subcore_axis_name="s")

    buf_set = [
        pltpu.VMEM((CHUNK, H), jnp.float32),
        pltpu.VMEM((CHUNK, UNI_LAYOUT), jnp.int32),
        pltpu.VMEM((CHUNK, UNI_LAYOUT), jnp.int32),
        pltpu.VMEM((CHUNK, H), jnp.float32),
        pltpu.SemaphoreType.DMA,
        pltpu.SemaphoreType.DMA,
        pltpu.SemaphoreType.DMA,
    ]

    @functools.partial(
        pl.kernel,
        mesh=mesh,
        out_type=jax.ShapeDtypeStruct((N, H), jnp.float32),
        compiler_params=pltpu.CompilerParams(needs_layout_passes=False),
        scratch_types=[
            pltpu.VMEM((PER_W,), jnp.int32),
            pltpu.VMEM((PER_W,), jnp.int32),
            pltpu.VMEM((PER_W,), jnp.int32),
            pltpu.VMEM((PER_W,), jnp.int32),
            pltpu.VMEM((PER_W,), jnp.int32),
            pltpu.VMEM((PER_W,), jnp.int32),
            pltpu.VMEM((PER_W,), jnp.int32),
            pltpu.VMEM((PER_W,), jnp.int32),
            pltpu.VMEM((TABS_LEN,), jnp.float32),
        ] + buf_set + buf_set,
    )
    def sc_kernel(tok_id, ord_id, mag_id, pre_id, top_id, low_id, row_id,
                  col_id, ptop, pleft, tokW, ordW, tabs, out_hbm,
                  i_tok, i_ord, i_mag, i_pre, i_top, i_low, i_row, i_col,
                  tv,
                  tok0, pt0, pl0, os0, gsem0, asem0, ssem0,
                  tok1, pt1, pl1, os1, gsem1, asem1, ssem1):
        wid = lax.axis_index("s") * NC + lax.axis_index("c")
        w0 = wid * PER_W
        br = w0 // S
        col0 = w0 % S
        pltpu.sync_copy(tabs, tv)
        pltpu.sync_copy(tok_id.at[br, pl.ds(col0, PER_W)], i_tok)
        pltpu.sync_copy(ord_id.at[br, pl.ds(col0, PER_W)], i_ord)
        pltpu.sync_copy(mag_id.at[br, pl.ds(col0, PER_W)], i_mag)
        pltpu.sync_copy(pre_id.at[br, pl.ds(col0, PER_W)], i_pre)
        pltpu.sync_copy(top_id.at[br, pl.ds(col0, PER_W)], i_top)
        pltpu.sync_copy(low_id.at[br, pl.ds(col0, PER_W)], i_low)
        pltpu.sync_copy(row_id.at[br, pl.ds(col0, PER_W)], i_row)
        pltpu.sync_copy(col_id.at[br, pl.ds(col0, PER_W)], i_col)

        bufs = ((tok0, pt0, pl0, os0, gsem0, asem0, ssem0),
                (tok1, pt1, pl1, os1, gsem1, asem1, ssem1))

        def gather_copies(c, bset):
            tokb, ptb, plb = bset[:3]
            gsem = bset[4]
            cc = col0 + c * CHUNK
            return (
                (tokW.at[i_tok.at[pl.ds(c * CHUNK, CHUNK)]], tokb, gsem),
                (ptop.at[br, pl.ds(cc, CHUNK), :], ptb, gsem),
                (pleft.at[br, pl.ds(cc, CHUNK), :], plb, gsem),
            )

        def issue(c, bset):
            for src, dst, sem in gather_copies(c, bset):
                pltpu.async_copy(src, dst, sem)

        def drain(c, bset):
            for src, dst, sem in gather_copies(c, bset):
                pltpu.make_async_copy(src, dst, sem).wait()

        def issue_add(c, bset):
            pltpu.async_copy(ordW.at[i_ord.at[pl.ds(c * CHUNK, CHUNK)]],
                             bset[3], bset[5])

        def drain_add(c, bset):
            pltpu.make_async_copy(
                ordW.at[i_ord.at[pl.ds(c * CHUNK, CHUNK)]],
                bset[3], bset[5]).wait()

        issue(0, bufs[0])
        issue(1, bufs[1])
        drain(0, bufs[0])
        issue_add(0, bufs[0])

        iota = jnp.arange(L, dtype=jnp.int32)

        def chunk_body(c, carry):
            for b in range(2):

                @pl.when(c % 2 == b)
                def _():
                    tokb, ptb, plb, osb, gsem, asem, ssem = bufs[b]
                    nset = bufs[1 - b]
                    drain_add(c, bufs[b])

                    half = (c // 2) * L
                    lane0 = (c % 2) * CHUNK
                    mg = i_mag[pl.ds(half, L)]
                    mp = i_pre[pl.ds(half, L)]
                    mt = i_top[pl.ds(half, L)]
                    ml = i_low[pl.ds(half, L)]
                    mr = i_row[pl.ds(half, L)]
                    mc = i_col[pl.ds(half, L)]

                    def pos_body(i, carry2):
                        lv = jnp.full((L,), lane0 + i, jnp.int32)
                        m_mag = _lane_bcast(mg, lv)
                        m_pre = _lane_bcast(mp, lv)
                        m_top = _lane_bcast(mt, lv)
                        m_low = _lane_bcast(ml, lv)
                        m_row = _lane_bcast(mr, lv)
                        m_col = _lane_bcast(mc, lv)
                        bases = (m_mag * NUM_EMB + iota + OFF_MAG,
                                 m_pre * NUM_EMB + iota + OFF_PRE,
                                 m_top * NUM_EMB + iota + OFF_TOP,
                                 m_low * NUM_EMB + iota + OFF_LOW)
                        b_row = m_row * UNI_LAYOUT + iota + OFF_ROW
                        b_col = m_col * UNI_LAYOUT + iota + OFF_COL
                        ptf = [ptb[i, pl.ds(k * L, L)].astype(jnp.float32)
                               for k in range(6)]
                        plf = [plb[i, pl.ds(k * L, L)].astype(jnp.float32)
                               for k in range(6)]
                        for g in range(G):
                            d = pl.ds(g * L, L)
                            x = tokb[i, d] + osb[i, d]
                            x = x + plsc.load_gather(
                                tv, [bases[g // 12] + (g % 12) * L])
                            if g < 6:
                                x = x + plsc.load_gather(tv, [b_row + g * L])
                            elif g < 24:
                                l0 = g * L - UNI_LAYOUT
                                x = x + (tv[pl.ds(OFF_TREE + UNI_TREE + l0, L)]
                                         * plf[(l0 // L) % 6])
                            elif g < 30:
                                x = x + plsc.load_gather(
                                    tv, [b_col + (g - 24) * L])
                            else:
                                l0 = g * L - 480
                                x = x + (tv[pl.ds(OFF_TREE + l0, L)]
                                         * ptf[(l0 // L) % 6])
                            osb[i, d] = x
                        return carry2

                    lax.fori_loop(0, CHUNK, pos_body, 0)
                    pltpu.async_copy(
                        osb, out_hbm.at[pl.ds(w0 + c * CHUNK, CHUNK), :],
                        ssem)

                    @pl.when(c < NCHUNK - 2)
                    def _():
                        issue(c + 2, bufs[b])

                    @pl.when(c < NCHUNK - 1)
                    def _():
                        drain(c + 1, nset)

                        @pl.when(c >= 1)
                        def _():
                            pltpu.make_async_copy(
                                nset[3], out_hbm.at[pl.ds(w0, CHUNK), :],
                                nset[6]).wait()

                        issue_add(c + 1, nset)

            return carry

        lax.fori_loop(0, NCHUNK, chunk_body, 0)
        for b in range(2):
            osb, ssem = bufs[b][3], bufs[b][6]
            pltpu.make_async_copy(
                osb, out_hbm.at[pl.ds(w0, CHUNK), :], ssem).wait()

    return sc_kernel


_SC_PARTIAL = _sc_partial()

TC_BLK = 512


def _tc_body(part_ref, fv_ref, fmtT_ref, g_ref, b_ref, o_ref):
    x = part_ref[...] + jnp.dot(fv_ref[...], fmtT_ref[...],
                                preferred_element_type=jnp.float32)
    mean = jnp.mean(x, axis=-1, keepdims=True)
    var = jnp.mean((x - mean) ** 2, axis=-1, keepdims=True)
    o_ref[...] = (x - mean) * lax.rsqrt(var + EPS) * g_ref[...] + b_ref[...]


def _tc_finish(partial, fv_pad, fmtT_pad, ln_g, ln_b):
    grid = (N // TC_BLK,)
    return pl.pallas_call(
        _tc_body,
        grid=grid,
        in_specs=[
            pl.BlockSpec((TC_BLK, H), lambda i: (i, 0)),
            pl.BlockSpec((TC_BLK, 16), lambda i: (i, 0)),
            pl.BlockSpec((16, H), lambda i: (0, 0)),
            pl.BlockSpec((H,), lambda i: (0,)),
            pl.BlockSpec((H,), lambda i: (0,)),
        ],
        out_specs=pl.BlockSpec((TC_BLK, H), lambda i: (i, 0)),
        out_shape=jax.ShapeDtypeStruct((N, H), jnp.float32),
    )(partial, fv_pad, fmtT_pad, ln_g, ln_b)


def kernel(token_id, num_mag, num_pre, num_top, num_low, order, pos_row,
           pos_col, pos_top, pos_left, format_vec, token_W, mag_W, pre_W,
           top_W, low_W, order_W, row_W, col_W, tree_W, fmt_W, ln_g, ln_b):
    i32 = jnp.int32
    tabs = jnp.concatenate([
        mag_W.ravel(), pre_W.ravel(), top_W.ravel(), low_W.ravel(),
        row_W.ravel(), col_W.ravel(), tree_W.ravel()])

    partial = _SC_PARTIAL(
        token_id.astype(i32), order.astype(i32), num_mag.astype(i32),
        num_pre.astype(i32), num_top.astype(i32), num_low.astype(i32),
        pos_row.astype(i32), pos_col.astype(i32),
        pos_top.astype(i32), pos_left.astype(i32),
        token_W, order_W, tabs)

    fv_pad = jnp.pad(format_vec.reshape(N, 11), ((0, 0), (0, 5)))
    fmtT_pad = jnp.pad(fmt_W.T, ((0, 5), (0, 0)))
    out = _tc_finish(partial, fv_pad, fmtT_pad, ln_g, ln_b)
    return out.reshape(B, S, H)


# speedup vs baseline: 1.2006x; 1.2006x over previous
import functools

import jax
import jax.numpy as jnp
from jax import lax
from jax.experimental import pallas as pl
from jax.experimental.pallas import tpu as pltpu
from jax.experimental.pallas import tpu_sc as plsc

B, S = 4, 2048
N = B * S
H = 768
NUM_EMB = H // 4
UNI_LAYOUT = NUM_EMB // 2
UNI_TREE = (H - NUM_EMB) // 2
EPS = 1e-6

NC, NS, L = 2, 16, 16
NW = NC * NS
PER_W = N // NW
CHUNK = 8
NCHUNK = PER_W // CHUNK
G = H // L

OFF_MAG = 0
OFF_PRE = OFF_MAG + 12 * NUM_EMB
OFF_TOP = OFF_PRE + 12 * NUM_EMB
OFF_LOW = OFF_TOP + 12 * NUM_EMB
OFF_ROW = OFF_LOW + 12 * NUM_EMB
OFF_COL = OFF_ROW + 257 * UNI_LAYOUT
OFF_TREE = OFF_COL + 257 * UNI_LAYOUT
TABS_LEN = OFF_TREE + 2 * UNI_TREE

_DNUMS = lax.GatherDimensionNumbers(
    offset_dims=(), collapsed_slice_dims=(0,), start_index_map=(0,))


def _lane_bcast(vec, idx):
    return lax.gather(vec, idx[:, None], _DNUMS, slice_sizes=(1,),
                      mode=lax.GatherScatterMode.PROMISE_IN_BOUNDS)


def _sc_partial():
    mesh = plsc.VectorSubcoreMesh(core_axis_name="c", subcore_axis_name="s")

    buf_set = [
        pltpu.VMEM((CHUNK, H), jnp.float32),
        pltpu.VMEM((CHUNK, H), jnp.float32),
        pltpu.VMEM((CHUNK, UNI_LAYOUT), jnp.int32),
        pltpu.VMEM((CHUNK, UNI_LAYOUT), jnp.int32),
        pltpu.VMEM((CHUNK, H), jnp.float32),
        pltpu.SemaphoreType.DMA,
        pltpu.SemaphoreType.DMA,
    ]

    @functools.partial(
        pl.kernel,
        mesh=mesh,
        out_type=jax.ShapeDtypeStruct((N, H), jnp.float32),
        compiler_params=pltpu.CompilerParams(needs_layout_passes=False),
        scratch_types=[
            pltpu.VMEM((PER_W,), jnp.int32),
            pltpu.VMEM((PER_W,), jnp.int32),
            pltpu.VMEM((PER_W,), jnp.int32),
            pltpu.VMEM((PER_W,), jnp.int32),
            pltpu.VMEM((PER_W,), jnp.int32),
            pltpu.VMEM((PER_W,), jnp.int32),
            pltpu.VMEM((PER_W,), jnp.int32),
            pltpu.VMEM((PER_W,), jnp.int32),
            pltpu.VMEM((TABS_LEN,), jnp.float32),
        ] + buf_set + buf_set,
    )
    def sc_kernel(tok_id, ord_id, mag_id, pre_id, top_id, low_id, row_id,
                  col_id, ptop, pleft, tokW, ordW, tabs, out_hbm,
                  i_tok, i_ord, i_mag, i_pre, i_top, i_low, i_row, i_col,
                  tv,
                  tok0, ord0, pt0, pl0, os0, gsem0, ssem0,
                  tok1, ord1, pt1, pl1, os1, gsem1, ssem1):
        wid = lax.axis_index("s") * NC + lax.axis_index("c")
        w0 = wid * PER_W
        br = w0 // S
        col0 = w0 % S
        pltpu.sync_copy(tabs, tv)
        pltpu.sync_copy(tok_id.at[br, pl.ds(col0, PER_W)], i_tok)
        pltpu.sync_copy(ord_id.at[br, pl.ds(col0, PER_W)], i_ord)
        pltpu.sync_copy(mag_id.at[br, pl.ds(col0, PER_W)], i_mag)
        pltpu.sync_copy(pre_id.at[br, pl.ds(col0, PER_W)], i_pre)
        pltpu.sync_copy(top_id.at[br, pl.ds(col0, PER_W)], i_top)
        pltpu.sync_copy(low_id.at[br, pl.ds(col0, PER_W)], i_low)
        pltpu.sync_copy(row_id.at[br, pl.ds(col0, PER_W)], i_row)
        pltpu.sync_copy(col_id.at[br, pl.ds(col0, PER_W)], i_col)

        bufs = ((tok0, ord0, pt0, pl0, os0, gsem0, ssem0),
                (tok1, ord1, pt1, pl1, os1, gsem1, ssem1))

        def gather_copies(c, bset):
            tokb, ordb, ptb, plb = bset[:4]
            gsem = bset[5]
            cc = col0 + c * CHUNK
            return (
                (tokW.at[i_tok.at[pl.ds(c * CHUNK, CHUNK)]], tokb, gsem),
                (ordW.at[i_ord.at[pl.ds(c * CHUNK, CHUNK)]], ordb, gsem),
                (ptop.at[br, pl.ds(cc, CHUNK), :], ptb, gsem),
                (pleft.at[br, pl.ds(cc, CHUNK), :], plb, gsem),
            )

        def issue(c, bset):
            for src, dst, sem in gather_copies(c, bset):
                pltpu.async_copy(src, dst, sem)

        def drain(c, bset):
            for src, dst, sem in gather_copies(c, bset):
                pltpu.make_async_copy(src, dst, sem).wait()

        issue(0, bufs[0])
        issue(1, bufs[1])

        iota = jnp.arange(L, dtype=jnp.int32)

        def chunk_body(c, carry):
            for b in range(2):

                @pl.when(c % 2 == b)
                def _():
                    tokb, ordb, ptb, plb, osb, gsem, ssem = bufs[b]
                    drain(c, bufs[b])

                    @pl.when(c >= 2)
                    def _():
                        pltpu.make_async_copy(
                            osb, out_hbm.at[pl.ds(w0, CHUNK), :], ssem).wait()

                    half = (c // 2) * L
                    lane0 = (c % 2) * CHUNK
                    mg = i_mag[pl.ds(half, L)]
                    mp = i_pre[pl.ds(half, L)]
                    mt = i_top[pl.ds(half, L)]
                    ml = i_low[pl.ds(half, L)]
                    mr = i_row[pl.ds(half, L)]
                    mc = i_col[pl.ds(half, L)]

                    def pos_body(i, carry2):
                        lv = jnp.full((L,), lane0 + i, jnp.int32)
                        m_mag = _lane_bcast(mg, lv)
                        m_pre = _lane_bcast(mp, lv)
                        m_top = _lane_bcast(mt, lv)
                        m_low = _lane_bcast(ml, lv)
                        m_row = _lane_bcast(mr, lv)
                        m_col = _lane_bcast(mc, lv)
                        bases = (m_mag * NUM_EMB + iota + OFF_MAG,
                                 m_pre * NUM_EMB + iota + OFF_PRE,
                                 m_top * NUM_EMB + iota + OFF_TOP,
                                 m_low * NUM_EMB + iota + OFF_LOW)
                        b_row = m_row * UNI_LAYOUT + iota + OFF_ROW
                        b_col = m_col * UNI_LAYOUT + iota + OFF_COL
                        ptf = [ptb[i, pl.ds(k * L, L)].astype(jnp.float32)
                               for k in range(6)]
                        plf = [plb[i, pl.ds(k * L, L)].astype(jnp.float32)
                               for k in range(6)]
                        for g in range(G):
                            d = pl.ds(g * L, L)
                            x = tokb[i, d] + ordb[i, d]
                            x = x + plsc.load_gather(
                                tv, [bases[g // 12] + (g % 12) * L])
                            if g < 6:
                                x = x + plsc.load_gather(tv, [b_row + g * L])
                            elif g < 24:
                                l0 = g * L - UNI_LAYOUT
                                x = x + (tv[pl.ds(OFF_TREE + UNI_TREE + l0, L)]
                                         * plf[(l0 // L) % 6])
                            elif g < 30:
                                x = x + plsc.load_gather(
                                    tv, [b_col + (g - 24) * L])
                            else:
                                l0 = g * L - 480
                                x = x + (tv[pl.ds(OFF_TREE + l0, L)]
                                         * ptf[(l0 // L) % 6])
                            osb[i, d] = x
                        return carry2

                    lax.fori_loop(0, CHUNK, pos_body, 0)
                    pltpu.async_copy(
                        osb, out_hbm.at[pl.ds(w0 + c * CHUNK, CHUNK), :],
                        ssem)

                    @pl.when(c < NCHUNK - 2)
                    def _():
                        issue(c + 2, bufs[b])

            return carry

        lax.fori_loop(0, NCHUNK, chunk_body, 0)
        for b in range(2):
            osb, ssem = bufs[b][4], bufs[b][6]
            pltpu.make_async_copy(
                osb, out_hbm.at[pl.ds(w0, CHUNK), :], ssem).wait()

    return sc_kernel


_SC_PARTIAL = _sc_partial()

TC_BLK = 512


def _tc_body(part_ref, fv_ref, fmtT_ref, g_ref, b_ref, o_ref):
    x = part_ref[...] + jnp.dot(fv_ref[...], fmtT_ref[...],
                                preferred_element_type=jnp.float32)
    mean = jnp.mean(x, axis=-1, keepdims=True)
    var = jnp.mean((x - mean) ** 2, axis=-1, keepdims=True)
    o_ref[...] = (x - mean) * lax.rsqrt(var + EPS) * g_ref[...] + b_ref[...]


def _tc_finish(partial, fv_pad, fmtT_pad, ln_g, ln_b):
    grid = (N // TC_BLK,)
    return pl.pallas_call(
        _tc_body,
        grid=grid,
        in_specs=[
            pl.BlockSpec((TC_BLK, H), lambda i: (i, 0)),
            pl.BlockSpec((TC_BLK, 16), lambda i: (i, 0)),
            pl.BlockSpec((16, H), lambda i: (0, 0)),
            pl.BlockSpec((H,), lambda i: (0,)),
            pl.BlockSpec((H,), lambda i: (0,)),
        ],
        out_specs=pl.BlockSpec((TC_BLK, H), lambda i: (i, 0)),
        out_shape=jax.ShapeDtypeStruct((N, H), jnp.float32),
    )(partial, fv_pad, fmtT_pad, ln_g, ln_b)


def kernel(token_id, num_mag, num_pre, num_top, num_low, order, pos_row,
           pos_col, pos_top, pos_left, format_vec, token_W, mag_W, pre_W,
           top_W, low_W, order_W, row_W, col_W, tree_W, fmt_W, ln_g, ln_b):
    i32 = jnp.int32
    tabs = jnp.concatenate([
        mag_W.ravel(), pre_W.ravel(), top_W.ravel(), low_W.ravel(),
        row_W.ravel(), col_W.ravel(), tree_W.ravel()])

    partial = _SC_PARTIAL(
        token_id.astype(i32), order.astype(i32), num_mag.astype(i32),
        num_pre.astype(i32), num_top.astype(i32), num_low.astype(i32),
        pos_row.astype(i32), pos_col.astype(i32),
        pos_top.astype(i32), pos_left.astype(i32),
        token_W, order_W, tabs)

    fv_pad = jnp.pad(format_vec.reshape(N, 11), ((0, 0), (0, 5)))
    fmtT_pad = jnp.pad(fmt_W.T, ((0, 5), (0, 0)))
    out = _tc_finish(partial, fv_pad, fmtT_pad, ln_g, ln_b)
    return out.reshape(B, S, H)


# speedup vs baseline: 1.3857x; 1.1542x over previous
import functools

import jax
import jax.numpy as jnp
from jax import lax
from jax.experimental import pallas as pl
from jax.experimental.pallas import tpu as pltpu
from jax.experimental.pallas import tpu_sc as plsc

B, S = 4, 2048
N = B * S
H = 768
NUM_EMB = H // 4
UNI_LAYOUT = NUM_EMB // 2
UNI_TREE = (H - NUM_EMB) // 2
EPS = 1e-6

NC, NS, L = 2, 16, 16
NW = NC * NS
PER_W = N // NW
CHUNK = 32
NBUF = 4
NCHUNK = PER_W // CHUNK


def _sc_gather():
    mesh = plsc.VectorSubcoreMesh(core_axis_name="c", subcore_axis_name="s")

    slot = [
        pltpu.VMEM((CHUNK, H), jnp.float32),
        pltpu.SemaphoreType.DMA,
        pltpu.SemaphoreType.DMA,
    ]

    @functools.partial(
        pl.kernel,
        mesh=mesh,
        out_type=jax.ShapeDtypeStruct((N, H), jnp.float32),
        compiler_params=pltpu.CompilerParams(needs_layout_passes=False),
        scratch_types=[pltpu.VMEM((PER_W,), jnp.int32)]
        + slot + slot + slot + slot,
    )
    def sc_kernel(tok_id, tokW, out_hbm, i_tok,
                  b0, g0, s0, b1, g1, s1, b2, g2, s2, b3, g3, s3):
        wid = lax.axis_index("s") * NC + lax.axis_index("c")
        w0 = wid * PER_W
        pltpu.sync_copy(tok_id.at[w0 // S, pl.ds(w0 % S, PER_W)], i_tok)

        bufs = ((b0, g0, s0), (b1, g1, s1), (b2, g2, s2), (b3, g3, s3))

        def gather(c, bset):
            return pltpu.make_async_copy(
                tokW.at[i_tok.at[pl.ds(c * CHUNK, CHUNK)]], bset[0], bset[1])

        def store(c, bset):
            return pltpu.make_async_copy(
                bset[0], out_hbm.at[pl.ds(w0 + c * CHUNK, CHUNK), :], bset[2])

        gather(0, bufs[0]).start()
        gather(1, bufs[1]).start()

        def chunk_body(c, carry):
            for b in range(NBUF):

                @pl.when(c % NBUF == b)
                def _():
                    gather(c, bufs[b]).wait()
                    store(c, bufs[b]).start()

                    b2i = (b + 2) % NBUF

                    @pl.when(c >= 2)
                    def _():
                        store(c - 2, bufs[b2i]).wait()

                    @pl.when(c + 2 < NCHUNK)
                    def _():
                        gather(c + 2, bufs[b2i]).start()

            return carry

        lax.fori_loop(0, NCHUNK, chunk_body, 0)
        store(NCHUNK - 2, bufs[(NCHUNK - 2) % NBUF]).wait()
        store(NCHUNK - 1, bufs[(NCHUNK - 1) % NBUF]).wait()

    return sc_kernel


_SC_GATHER = _sc_gather()

TC_BLK = 512


def _onehot(ids, n):
    return (ids[:, None]
            == lax.broadcasted_iota(jnp.int32, (TC_BLK, n), 1)
            ).astype(jnp.float32)


def _tc_body(part_ref, ord_ref, mag_ref, pre_ref, top_ref, low_ref,
             row_ref, col_ref, pt_ref, pl_ref, fv_ref,
             ordW_ref, magW_ref, preW_ref, topW_ref, lowW_ref,
             rowW_ref, colW_ref, treeW_ref, fmtW_ref, g_ref, b_ref, o_ref):
    f32 = jnp.float32
    bi = pl.program_id(0)
    sj = pl.ds(pl.program_id(1) * TC_BLK, TC_BLK)
    dot = functools.partial(jnp.dot, preferred_element_type=f32,
                            precision=lax.Precision.HIGHEST)
    numeric = jnp.concatenate(
        [dot(_onehot(mag_ref[bi, sj], 12), magW_ref[...]),
         dot(_onehot(pre_ref[bi, sj], 12), preW_ref[...]),
         dot(_onehot(top_ref[bi, sj], 12), topW_ref[...]),
         dot(_onehot(low_ref[bi, sj], 12), lowW_ref[...])], axis=1)
    order_states = dot(_onehot(ord_ref[bi, sj], 256), ordW_ref[...])
    row_states = dot(_onehot(row_ref[bi, sj], 257), rowW_ref[...])
    col_states = dot(_onehot(col_ref[bi, sj], 257), colW_ref[...])
    ptf = pt_ref[0].astype(f32)
    plf = pl_ref[0].astype(f32)
    top_tree = jnp.tile(ptf, (1, 3)) * treeW_ref[0][None, :]
    left_tree = jnp.tile(plf, (1, 3)) * treeW_ref[1][None, :]
    position = order_states + jnp.concatenate(
        [row_states, left_tree, col_states, top_tree], axis=1)
    fmt_states = lax.dot_general(
        fv_ref[0], fmtW_ref[...], (((1,), (1,)), ((), ())),
        preferred_element_type=f32, precision=lax.Precision.HIGHEST)
    x = part_ref[...] + numeric + position + fmt_states
    mean = jnp.mean(x, axis=-1, keepdims=True)
    var = jnp.mean((x - mean) ** 2, axis=-1, keepdims=True)
    o_ref[0] = ((x - mean) * lax.rsqrt(var + EPS) * g_ref[...]
                + b_ref[...])


def _tc_finish(partial, order, num_mag, num_pre, num_top, num_low,
               pos_row, pos_col, pos_top, pos_left, format_vec,
               order_W, mag_W, pre_W, top_W, low_W, row_W, col_W,
               tree_W, fmt_W, ln_g, ln_b):
    grid = (B, S // TC_BLK)
    ids_spec = pl.BlockSpec((B, S), lambda i, j: (0, 0))
    pos_spec = pl.BlockSpec((1, TC_BLK, UNI_LAYOUT), lambda i, j: (i, j, 0))
    full = lambda *shape: pl.BlockSpec(shape, lambda i, j: (0,) * len(shape))
    return pl.pallas_call(
        _tc_body,
        grid=grid,
        in_specs=[
            pl.BlockSpec((TC_BLK, H),
                         lambda i, j: (i * (S // TC_BLK) + j, 0)),
            ids_spec, ids_spec, ids_spec, ids_spec, ids_spec,
            ids_spec, ids_spec, pos_spec, pos_spec,
            pl.BlockSpec((1, TC_BLK, 11), lambda i, j: (i, j, 0)),
            full(256, H), full(12, NUM_EMB), full(12, NUM_EMB),
            full(12, NUM_EMB), full(12, NUM_EMB),
            full(257, UNI_LAYOUT), full(257, UNI_LAYOUT),
            full(2, UNI_TREE), full(H, 11), full(H,), full(H,),
        ],
        out_specs=pl.BlockSpec((1, TC_BLK, H), lambda i, j: (i, j, 0)),
        out_shape=jax.ShapeDtypeStruct((B, S, H), jnp.float32),
    )(partial, order, num_mag, num_pre, num_top, num_low,
      pos_row, pos_col, pos_top, pos_left, format_vec,
      order_W, mag_W, pre_W, top_W, low_W, row_W, col_W,
      tree_W, fmt_W, ln_g, ln_b)


def kernel(token_id, num_mag, num_pre, num_top, num_low, order, pos_row,
           pos_col, pos_top, pos_left, format_vec, token_W, mag_W, pre_W,
           top_W, low_W, order_W, row_W, col_W, tree_W, fmt_W, ln_g, ln_b):
    i32 = jnp.int32
    partial = _SC_GATHER(token_id.astype(i32), token_W)
    return _tc_finish(partial, order.astype(i32), num_mag.astype(i32),
                      num_pre.astype(i32), num_top.astype(i32),
                      num_low.astype(i32), pos_row.astype(i32),
                      pos_col.astype(i32), pos_top.astype(i32),
                      pos_left.astype(i32), format_vec,
                      order_W, mag_W, pre_W, top_W, low_W, row_W, col_W,
                      tree_W, fmt_W, ln_g, ln_b)


# speedup vs baseline: 1.9813x; 1.4298x over previous
import functools

import jax
import jax.numpy as jnp
from jax import lax
from jax.experimental import pallas as pl
from jax.experimental.pallas import tpu as pltpu
from jax.experimental.pallas import tpu_sc as plsc

B, S = 4, 2048
N = B * S
H = 768
NUM_EMB = H // 4
UNI_LAYOUT = NUM_EMB // 2
UNI_TREE = (H - NUM_EMB) // 2
EPS = 1e-6

NC, NS, L = 2, 16, 16
NW = NC * NS
PER_W = N // NW
CHUNK = 32
NBUF = 4
NCHUNK = PER_W // CHUNK


def _sc_gather():
    mesh = plsc.VectorSubcoreMesh(core_axis_name="c", subcore_axis_name="s")

    slot = [
        pltpu.VMEM((CHUNK, H), jnp.float32),
        pltpu.SemaphoreType.DMA,
        pltpu.SemaphoreType.DMA,
    ]

    @functools.partial(
        pl.kernel,
        mesh=mesh,
        out_type=jax.ShapeDtypeStruct((N, H), jnp.float32),
        compiler_params=pltpu.CompilerParams(needs_layout_passes=False),
        scratch_types=[pltpu.VMEM((PER_W,), jnp.int32)]
        + slot + slot + slot + slot,
    )
    def sc_kernel(tok_id, tokW, out_hbm, i_tok,
                  b0, g0, s0, b1, g1, s1, b2, g2, s2, b3, g3, s3):
        wid = lax.axis_index("s") * NC + lax.axis_index("c")
        w0 = wid * PER_W
        pltpu.sync_copy(tok_id.at[w0 // S, pl.ds(w0 % S, PER_W)], i_tok)

        bufs = ((b0, g0, s0), (b1, g1, s1), (b2, g2, s2), (b3, g3, s3))

        def gather(c, bset):
            return pltpu.make_async_copy(
                tokW.at[i_tok.at[pl.ds(c * CHUNK, CHUNK)]], bset[0], bset[1])

        def store(c, bset):
            return pltpu.make_async_copy(
                bset[0], out_hbm.at[pl.ds(w0 + c * CHUNK, CHUNK), :], bset[2])

        gather(0, bufs[0]).start()
        gather(1, bufs[1]).start()

        def chunk_body(c, carry):
            for b in range(NBUF):

                @pl.when(c % NBUF == b)
                def _():
                    gather(c, bufs[b]).wait()
                    store(c, bufs[b]).start()

                    b2i = (b + 2) % NBUF

                    @pl.when(c >= 2)
                    def _():
                        store(c - 2, bufs[b2i]).wait()

                    @pl.when(c + 2 < NCHUNK)
                    def _():
                        gather(c + 2, bufs[b2i]).start()

            return carry

        lax.fori_loop(0, NCHUNK, chunk_body, 0)
        store(NCHUNK - 2, bufs[(NCHUNK - 2) % NBUF]).wait()
        store(NCHUNK - 1, bufs[(NCHUNK - 1) % NBUF]).wait()

    return sc_kernel


_SC_GATHER = _sc_gather()

TC_BLK = 512


def _onehot(ids, n):
    return (ids[:, None]
            == lax.broadcasted_iota(jnp.int32, (TC_BLK, n), 1)
            ).astype(jnp.float32)


def _select(oh, w_ref):
    w = w_ref[...]
    hi = w.astype(jnp.bfloat16).astype(jnp.float32)
    lo = w - hi
    d = functools.partial(jnp.dot, preferred_element_type=jnp.float32)
    return d(oh, hi) + d(oh, lo)


def _tc_body(part_ref, ord_ref, mag_ref, pre_ref, top_ref, low_ref,
             row_ref, col_ref, pt_ref, pl_ref, fv_ref,
             ordW_ref, magW_ref, preW_ref, topW_ref, lowW_ref,
             rowW_ref, colW_ref, treeW_ref, fmtW_ref, g_ref, b_ref, o_ref):
    f32 = jnp.float32
    bi = pl.program_id(0)
    sj = pl.ds(pl.program_id(1) * TC_BLK, TC_BLK)
    numeric = jnp.concatenate(
        [_select(_onehot(mag_ref[bi, sj], 12), magW_ref),
         _select(_onehot(pre_ref[bi, sj], 12), preW_ref),
         _select(_onehot(top_ref[bi, sj], 12), topW_ref),
         _select(_onehot(low_ref[bi, sj], 12), lowW_ref)], axis=1)
    order_states = _select(_onehot(ord_ref[bi, sj], 256), ordW_ref)
    row_states = _select(_onehot(row_ref[bi, sj], 257), rowW_ref)
    col_states = _select(_onehot(col_ref[bi, sj], 257), colW_ref)
    ptf = pt_ref[0].astype(f32)
    plf = pl_ref[0].astype(f32)
    top_tree = jnp.tile(ptf, (1, 3)) * treeW_ref[0][None, :]
    left_tree = jnp.tile(plf, (1, 3)) * treeW_ref[1][None, :]
    position = order_states + jnp.concatenate(
        [row_states, left_tree, col_states, top_tree], axis=1)
    fmt_states = lax.dot_general(
        fv_ref[0], fmtW_ref[...], (((1,), (1,)), ((), ())),
        preferred_element_type=f32, precision=lax.Precision.HIGHEST)
    x = part_ref[...] + numeric + position + fmt_states
    mean = jnp.mean(x, axis=-1, keepdims=True)
    var = jnp.mean((x - mean) ** 2, axis=-1, keepdims=True)
    o_ref[0] = ((x - mean) * lax.rsqrt(var + EPS) * g_ref[...]
                + b_ref[...])


def _tc_finish(partial, order, num_mag, num_pre, num_top, num_low,
               pos_row, pos_col, pos_top, pos_left, format_vec,
               order_W, mag_W, pre_W, top_W, low_W, row_W, col_W,
               tree_W, fmt_W, ln_g, ln_b):
    grid = (B, S // TC_BLK)
    ids_spec = pl.BlockSpec((B, S), lambda i, j: (0, 0))
    pos_spec = pl.BlockSpec((1, TC_BLK, UNI_LAYOUT), lambda i, j: (i, j, 0))
    full = lambda *shape: pl.BlockSpec(shape, lambda i, j: (0,) * len(shape))
    return pl.pallas_call(
        _tc_body,
        grid=grid,
        in_specs=[
            pl.BlockSpec((TC_BLK, H),
                         lambda i, j: (i * (S // TC_BLK) + j, 0)),
            ids_spec, ids_spec, ids_spec, ids_spec, ids_spec,
            ids_spec, ids_spec, pos_spec, pos_spec,
            pl.BlockSpec((1, TC_BLK, 11), lambda i, j: (i, j, 0)),
            full(256, H), full(12, NUM_EMB), full(12, NUM_EMB),
            full(12, NUM_EMB), full(12, NUM_EMB),
            full(257, UNI_LAYOUT), full(257, UNI_LAYOUT),
            full(2, UNI_TREE), full(H, 11), full(H,), full(H,),
        ],
        out_specs=pl.BlockSpec((1, TC_BLK, H), lambda i, j: (i, j, 0)),
        out_shape=jax.ShapeDtypeStruct((B, S, H), jnp.float32),
    )(partial, order, num_mag, num_pre, num_top, num_low,
      pos_row, pos_col, pos_top, pos_left, format_vec,
      order_W, mag_W, pre_W, top_W, low_W, row_W, col_W,
      tree_W, fmt_W, ln_g, ln_b)


def kernel(token_id, num_mag, num_pre, num_top, num_low, order, pos_row,
           pos_col, pos_top, pos_left, format_vec, token_W, mag_W, pre_W,
           top_W, low_W, order_W, row_W, col_W, tree_W, fmt_W, ln_g, ln_b):
    i32 = jnp.int32
    partial = _SC_GATHER(token_id.astype(i32), token_W)
    return _tc_finish(partial, order.astype(i32), num_mag.astype(i32),
                      num_pre.astype(i32), num_top.astype(i32),
                      num_low.astype(i32), pos_row.astype(i32),
                      pos_col.astype(i32), pos_top.astype(i32),
                      pos_left.astype(i32), format_vec,
                      order_W, mag_W, pre_W, top_W, low_W, row_W, col_W,
                      tree_W, fmt_W, ln_g, ln_b)


# speedup vs baseline: 2.2306x; 1.1258x over previous
import functools

import jax
import jax.numpy as jnp
from jax import lax
from jax.experimental import pallas as pl
from jax.experimental.pallas import tpu as pltpu
from jax.experimental.pallas import tpu_sc as plsc

B, S = 4, 2048
N = B * S
H = 768
NUM_EMB = H // 4
UNI_LAYOUT = NUM_EMB // 2
UNI_TREE = (H - NUM_EMB) // 2
EPS = 1e-6

NC, NS, L = 2, 16, 16
NW = NC * NS
PER_W = N // NW
CHUNK = 32
NBUF = 4
NCHUNK = PER_W // CHUNK


def _sc_gather():
    mesh = plsc.VectorSubcoreMesh(core_axis_name="c", subcore_axis_name="s")

    slot = [
        pltpu.VMEM((CHUNK, H), jnp.float32),
        pltpu.SemaphoreType.DMA,
        pltpu.SemaphoreType.DMA,
    ]

    @functools.partial(
        pl.kernel,
        mesh=mesh,
        out_type=jax.ShapeDtypeStruct((N, H), jnp.float32),
        compiler_params=pltpu.CompilerParams(needs_layout_passes=False),
        scratch_types=[pltpu.VMEM((PER_W,), jnp.int32)]
        + slot + slot + slot + slot,
    )
    def sc_kernel(tok_id, tokW, out_hbm, i_tok,
                  b0, g0, s0, b1, g1, s1, b2, g2, s2, b3, g3, s3):
        wid = lax.axis_index("s") * NC + lax.axis_index("c")
        w0 = wid * PER_W
        pltpu.sync_copy(tok_id.at[w0 // S, pl.ds(w0 % S, PER_W)], i_tok)

        bufs = ((b0, g0, s0), (b1, g1, s1), (b2, g2, s2), (b3, g3, s3))

        def gather(c, bset):
            return pltpu.make_async_copy(
                tokW.at[i_tok.at[pl.ds(c * CHUNK, CHUNK)]], bset[0], bset[1])

        def store(c, bset):
            return pltpu.make_async_copy(
                bset[0], out_hbm.at[pl.ds(w0 + c * CHUNK, CHUNK), :], bset[2])

        gather(0, bufs[0]).start()
        gather(1, bufs[1]).start()

        def chunk_body(c, carry):
            for b in range(NBUF):

                @pl.when(c % NBUF == b)
                def _():
                    gather(c, bufs[b]).wait()
                    store(c, bufs[b]).start()

                    b2i = (b + 2) % NBUF

                    @pl.when(c >= 2)
                    def _():
                        store(c - 2, bufs[b2i]).wait()

                    @pl.when(c + 2 < NCHUNK)
                    def _():
                        gather(c + 2, bufs[b2i]).start()

            return carry

        lax.fori_loop(0, NCHUNK, chunk_body, 0)
        store(NCHUNK - 2, bufs[(NCHUNK - 2) % NBUF]).wait()
        store(NCHUNK - 1, bufs[(NCHUNK - 1) % NBUF]).wait()

    return sc_kernel


_SC_GATHER = _sc_gather()

TC_BLK = 512


def _onehot(ids, n):
    return (ids[:, None]
            == lax.broadcasted_iota(jnp.int32, (TC_BLK, n), 1)
            ).astype(jnp.float32)


def _select(oh, w_ref):
    w = w_ref[...]
    hi = w.astype(jnp.bfloat16).astype(jnp.float32)
    lo = w - hi
    d = functools.partial(jnp.dot, preferred_element_type=jnp.float32)
    return d(oh, hi) + d(oh, lo)


def _tc_body(part_ref, ord_ref, mag_ref, pre_ref, top_ref, low_ref,
             row_ref, col_ref, pt_ref, pl_ref, fv_ref,
             ordW_ref, magW_ref, preW_ref, topW_ref, lowW_ref,
             rowW_ref, colW_ref, treeW_ref, fmtW_ref, g_ref, b_ref, o_ref):
    f32 = jnp.float32
    bi = pl.program_id(0)
    sj = pl.ds(pl.program_id(1) * TC_BLK, TC_BLK)
    numeric = jnp.concatenate(
        [_select(_onehot(mag_ref[bi, sj], 12), magW_ref),
         _select(_onehot(pre_ref[bi, sj], 12), preW_ref),
         _select(_onehot(top_ref[bi, sj], 12), topW_ref),
         _select(_onehot(low_ref[bi, sj], 12), lowW_ref)], axis=1)
    order_states = _select(_onehot(ord_ref[bi, sj], 256), ordW_ref)
    row_states = _select(_onehot(row_ref[bi, sj], 257), rowW_ref)
    col_states = _select(_onehot(col_ref[bi, sj], 257), colW_ref)
    ptf = pt_ref[0].astype(f32)
    plf = pl_ref[0].astype(f32)
    top_tree = jnp.tile(ptf, (1, 3)) * treeW_ref[0][None, :]
    left_tree = jnp.tile(plf, (1, 3)) * treeW_ref[1][None, :]
    position = order_states + jnp.concatenate(
        [row_states, left_tree, col_states, top_tree], axis=1)
    fv = fv_ref[0]
    fv_hi = fv.astype(jnp.bfloat16).astype(f32)
    fv_lo = fv - fv_hi
    fw = fmtW_ref[...]
    fw_hi = fw.astype(jnp.bfloat16).astype(f32)
    fw_lo = fw - fw_hi
    dg = functools.partial(
        lax.dot_general, dimension_numbers=(((1,), (1,)), ((), ())),
        preferred_element_type=f32)
    fmt_states = dg(fv_hi, fw_hi) + dg(fv_hi, fw_lo) + dg(fv_lo, fw_hi)
    x = part_ref[...] + numeric + position + fmt_states
    mean = jnp.mean(x, axis=-1, keepdims=True)
    var = jnp.mean((x - mean) ** 2, axis=-1, keepdims=True)
    o_ref[0] = ((x - mean) * lax.rsqrt(var + EPS) * g_ref[...]
                + b_ref[...])


def _tc_finish(partial, order, num_mag, num_pre, num_top, num_low,
               pos_row, pos_col, pos_top, pos_left, format_vec,
               order_W, mag_W, pre_W, top_W, low_W, row_W, col_W,
               tree_W, fmt_W, ln_g, ln_b):
    grid = (B, S // TC_BLK)
    ids_spec = pl.BlockSpec((B, S), lambda i, j: (0, 0))
    pos_spec = pl.BlockSpec((1, TC_BLK, UNI_LAYOUT), lambda i, j: (i, j, 0))
    full = lambda *shape: pl.BlockSpec(shape, lambda i, j: (0,) * len(shape))
    return pl.pallas_call(
        _tc_body,
        grid=grid,
        in_specs=[
            pl.BlockSpec((TC_BLK, H),
                         lambda i, j: (i * (S // TC_BLK) + j, 0)),
            ids_spec, ids_spec, ids_spec, ids_spec, ids_spec,
            ids_spec, ids_spec, pos_spec, pos_spec,
            pl.BlockSpec((1, TC_BLK, 11), lambda i, j: (i, j, 0)),
            full(256, H), full(12, NUM_EMB), full(12, NUM_EMB),
            full(12, NUM_EMB), full(12, NUM_EMB),
            full(257, UNI_LAYOUT), full(257, UNI_LAYOUT),
            full(2, UNI_TREE), full(H, 11), full(H,), full(H,),
        ],
        out_specs=pl.BlockSpec((1, TC_BLK, H), lambda i, j: (i, j, 0)),
        out_shape=jax.ShapeDtypeStruct((B, S, H), jnp.float32),
    )(partial, order, num_mag, num_pre, num_top, num_low,
      pos_row, pos_col, pos_top, pos_left, format_vec,
      order_W, mag_W, pre_W, top_W, low_W, row_W, col_W,
      tree_W, fmt_W, ln_g, ln_b)


def kernel(token_id, num_mag, num_pre, num_top, num_low, order, pos_row,
           pos_col, pos_top, pos_left, format_vec, token_W, mag_W, pre_W,
           top_W, low_W, order_W, row_W, col_W, tree_W, fmt_W, ln_g, ln_b):
    i32 = jnp.int32
    partial = _SC_GATHER(token_id.astype(i32), token_W)
    return _tc_finish(partial, order.astype(i32), num_mag.astype(i32),
                      num_pre.astype(i32), num_top.astype(i32),
                      num_low.astype(i32), pos_row.astype(i32),
                      pos_col.astype(i32), pos_top.astype(i32),
                      pos_left.astype(i32), format_vec,
                      order_W, mag_W, pre_W, top_W, low_W, row_W, col_W,
                      tree_W, fmt_W, ln_g, ln_b)
